# Initial kernel scaffold; baseline (speedup 1.0000x reference)
import jax
import jax.numpy as jnp
from jax.experimental import pallas as pl


def kernel(x, edge_index, edge_weight):
    raise NotImplementedError("write your pallas kernel here")


# speedup vs baseline: 1.2845x; 1.2845x over previous
import functools

import jax
import jax.numpy as jnp
from jax import lax
from jax.experimental import pallas as pl
from jax.experimental.pallas import tpu as pltpu
from jax.experimental.pallas import tpu_sc as plsc

N_NODES = 10000
N_EDGES = 160000
D = 256

NC = 2
NS = 16
L = 16
G = D // L

CHUNK = 64
PEND = 192
SUPER = 1024
NSUP = 160
E_PAD = NSUP * SUPER
GPB = SUPER // L

ROWS_PT = 320


def _sc_body(x_hbm, src_hbm, dst_hbm, w_hbm, out_hbm,
             acc, sup_src, sup_dst, sup_w, pend_src, pend_w, pend_loc,
             rows1, gsem, tsem0, tsem1):
    c = lax.axis_index("c")
    s = lax.axis_index("s")
    tsem = (tsem0, tsem1)
    t = c * NS + s
    base = t * ROWS_PT
    zeros = jnp.zeros((L,), jnp.float32)
    iota = lax.iota(jnp.int32, L)

    def staging_cps(q, sb):
        e0 = q * SUPER
        return (
            pltpu.make_async_copy(src_hbm.at[pl.ds(e0, SUPER)],
                                  sup_src.at[sb], tsem[sb]),
            pltpu.make_async_copy(dst_hbm.at[pl.ds(e0, SUPER)],
                                  sup_dst.at[sb], tsem[sb]),
            pltpu.make_async_copy(w_hbm.at[pl.ds(e0, SUPER)],
                                  sup_w.at[sb], tsem[sb]),
        )

    def fire_staging(q1):
        @pl.when((q1 & 1) == 0)
        def _():
            for cp in staging_cps(q1, 0):
                cp.start()

        @pl.when((q1 & 1) == 1)
        def _():
            for cp in staging_cps(q1, 1):
                cp.start()

    def wait_staging(q1):
        @pl.when((q1 & 1) == 0)
        def _():
            for cp in staging_cps(q1, 0):
                cp.wait()

        @pl.when((q1 & 1) == 1)
        def _():
            for cp in staging_cps(q1, 1):
                cp.wait()

    def process_chunk():
        pltpu.async_copy(x_hbm.at[pend_src.at[pl.ds(0, CHUNK)]],
                         rows1, gsem).wait()

        @pl.loop(0, CHUNK // L)
        def _grp(g):
            go = pl.multiple_of(g * L, L)
            w16 = pend_w[pl.ds(go, L)]
            l16 = pend_loc[pl.ds(go, L)]
            for k in range(L):
                wv = jnp.full((L,), w16[k], dtype=jnp.float32)
                lk = l16[k]
                e = go + k
                for j in range(G):
                    acc[lk, pl.ds(j * L, L)] = (
                        acc[lk, pl.ds(j * L, L)]
                        + rows1[e, pl.ds(j * L, L)] * wv)

        for g in range((PEND - CHUNK) // L):
            o_src = CHUNK + g * L
            o_dst = g * L
            pend_src[pl.ds(o_dst, L)] = pend_src[pl.ds(o_src, L)]
            pend_w[pl.ds(o_dst, L)] = pend_w[pl.ds(o_src, L)]
            pend_loc[pl.ds(o_dst, L)] = pend_loc[pl.ds(o_src, L)]

    @pl.loop(0, ROWS_PT)
    def _zero(r):
        for j in range(G):
            acc[r, pl.ds(j * L, L)] = zeros

    for cp in staging_cps(0, 0):
        cp.start()
    for cp in staging_cps(0, 0):
        cp.wait()

    @pl.loop(0, NSUP, init_carry=jnp.int32(0))
    def _block(q, pos_in):
        @pl.when(q + 1 < NSUP)
        def _():
            fire_staging(q + 1)

        sb = q & 1

        @pl.loop(0, GPB, init_carry=pos_in)
        def _grp(gi, pos):
            off = pl.multiple_of(gi * L, L)
            d16 = sup_dst[sb, pl.ds(off, L)]
            l16 = d16 - base
            m = jnp.logical_and(l16 >= 0, l16 < ROWS_PT)
            cnt = plsc.all_reduce_population_count(m)

            @pl.when(cnt[0] > 0)
            def _():
                s16 = sup_src[sb, pl.ds(off, L)]
                w16 = sup_w[sb, pl.ds(off, L)]
                excl = plsc.cumsum(m.astype(jnp.int32)) - m.astype(jnp.int32)
                slot = pos + excl
                plsc.store_scatter(pend_src, [slot], s16, mask=m)
                plsc.store_scatter(pend_w, [slot], w16, mask=m)
                plsc.store_scatter(pend_loc, [slot], l16, mask=m)

            pos2 = pos + cnt[0]

            @pl.when(pos2 >= CHUNK)
            def _():
                process_chunk()

            return jnp.where(pos2 >= CHUNK, pos2 - CHUNK, pos2)

        @pl.when(q + 1 < NSUP)
        def _():
            wait_staging(q + 1)

        return _grp

    pos_end = _block

    for g in range(CHUNK // L):
        idx = jnp.full((L,), g * L, jnp.int32) + iota
        m = idx >= pos_end
        plsc.store_scatter(pend_w, [idx], jnp.zeros((L,), jnp.float32),
                           mask=m)
        plsc.store_scatter(pend_loc, [idx], jnp.zeros((L,), jnp.int32),
                           mask=m)
        plsc.store_scatter(pend_src, [idx], jnp.zeros((L,), jnp.int32),
                           mask=m)
    process_chunk()

    last = NC * NS - 1

    @pl.when(t < last)
    def _():
        pltpu.sync_copy(acc, out_hbm.at[pl.ds(base, ROWS_PT)])

    @pl.when(t == last)
    def _():
        n = N_NODES - last * ROWS_PT
        pltpu.sync_copy(acc.at[pl.ds(0, n)], out_hbm.at[pl.ds(base, n)])


_MESH = plsc.VectorSubcoreMesh(core_axis_name="c", subcore_axis_name="s")


@functools.partial(
    pl.kernel,
    out_type=jax.ShapeDtypeStruct((N_NODES, D), jnp.float32),
    mesh=_MESH,
    compiler_params=pltpu.CompilerParams(needs_layout_passes=False),
    scratch_types=[
        pltpu.VMEM((ROWS_PT, D), jnp.float32),
        pltpu.VMEM((2, SUPER), jnp.int32),
        pltpu.VMEM((2, SUPER), jnp.int32),
        pltpu.VMEM((2, SUPER), jnp.float32),
        pltpu.VMEM((PEND,), jnp.int32),
        pltpu.VMEM((PEND,), jnp.float32),
        pltpu.VMEM((PEND,), jnp.int32),
        pltpu.VMEM((CHUNK, D), jnp.float32),
        pltpu.SemaphoreType.DMA,
        pltpu.SemaphoreType.DMA,
        pltpu.SemaphoreType.DMA,
    ],
)
def _hyperbolic_agg_sc(x_hbm, src_hbm, dst_hbm, w_hbm, out_hbm, *scratch):
    _sc_body(x_hbm, src_hbm, dst_hbm, w_hbm, out_hbm, *scratch)


def kernel(x, edge_index, edge_weight):
    src = edge_index[0].astype(jnp.int32)
    dst = edge_index[1].astype(jnp.int32)
    w = edge_weight.astype(jnp.float32)
    pad = E_PAD - src.shape[0]
    src = jnp.concatenate([src, jnp.zeros((pad,), jnp.int32)])
    dst = jnp.concatenate([dst, jnp.full((pad,), NC * NS * ROWS_PT,
                                         jnp.int32)])
    w = jnp.concatenate([w, jnp.zeros((pad,), jnp.float32)])
    return _hyperbolic_agg_sc(x, src, dst, w)


# speedup vs baseline: 1.3248x; 1.0313x over previous
import functools

import jax
import jax.numpy as jnp
from jax import lax
from jax.experimental import pallas as pl
from jax.experimental.pallas import tpu as pltpu
from jax.experimental.pallas import tpu_sc as plsc

N_NODES = 10000
N_EDGES = 160000
D = 256

NC = 2
NS = 16
L = 16
G = D // L

CHUNK = 128
PEND = 192
SUPER = 1024
NSUP = 160
E_PAD = NSUP * SUPER
GPB = SUPER // L

ROWS_PT = 320


def _sc_body(x_hbm, src_hbm, dst_hbm, w_hbm, out_hbm,
             acc, sup_src, sup_dst, sup_w, pend_src, pend_w, pend_loc,
             rows1, gsem, tsem0, tsem1):
    c = lax.axis_index("c")
    s = lax.axis_index("s")
    tsem = (tsem0, tsem1)
    t = c * NS + s
    base = t * ROWS_PT
    zeros = jnp.zeros((L,), jnp.float32)
    iota = lax.iota(jnp.int32, L)

    def staging_cps(q, sb):
        e0 = q * SUPER
        return (
            pltpu.make_async_copy(src_hbm.at[pl.ds(e0, SUPER)],
                                  sup_src.at[sb], tsem[sb]),
            pltpu.make_async_copy(dst_hbm.at[pl.ds(e0, SUPER)],
                                  sup_dst.at[sb], tsem[sb]),
            pltpu.make_async_copy(w_hbm.at[pl.ds(e0, SUPER)],
                                  sup_w.at[sb], tsem[sb]),
        )

    def fire_staging(q1):
        @pl.when((q1 & 1) == 0)
        def _():
            for cp in staging_cps(q1, 0):
                cp.start()

        @pl.when((q1 & 1) == 1)
        def _():
            for cp in staging_cps(q1, 1):
                cp.start()

    def wait_staging(q1):
        @pl.when((q1 & 1) == 0)
        def _():
            for cp in staging_cps(q1, 0):
                cp.wait()

        @pl.when((q1 & 1) == 1)
        def _():
            for cp in staging_cps(q1, 1):
                cp.wait()

    def process_chunk():
        pltpu.async_copy(x_hbm.at[pend_src.at[pl.ds(0, CHUNK)]],
                         rows1, gsem).wait()

        @pl.loop(0, CHUNK // L)
        def _grp(g):
            go = pl.multiple_of(g * L, L)
            w16 = pend_w[pl.ds(go, L)]
            l16 = pend_loc[pl.ds(go, L)]
            for k in range(L):
                wv = jnp.full((L,), w16[k], dtype=jnp.float32)
                lk = l16[k]
                e = go + k
                for j in range(G):
                    acc[lk, pl.ds(j * L, L)] = (
                        acc[lk, pl.ds(j * L, L)]
                        + rows1[e, pl.ds(j * L, L)] * wv)

        for g in range((PEND - CHUNK) // L):
            o_src = CHUNK + g * L
            o_dst = g * L
            pend_src[pl.ds(o_dst, L)] = pend_src[pl.ds(o_src, L)]
            pend_w[pl.ds(o_dst, L)] = pend_w[pl.ds(o_src, L)]
            pend_loc[pl.ds(o_dst, L)] = pend_loc[pl.ds(o_src, L)]

    @pl.loop(0, ROWS_PT)
    def _zero(r):
        for j in range(G):
            acc[r, pl.ds(j * L, L)] = zeros

    for cp in staging_cps(0, 0):
        cp.start()
    for cp in staging_cps(0, 0):
        cp.wait()

    @pl.loop(0, NSUP, init_carry=jnp.int32(0))
    def _block(q, pos_in):
        @pl.when(q + 1 < NSUP)
        def _():
            fire_staging(q + 1)

        sb = q & 1

        @pl.loop(0, GPB, init_carry=pos_in, unroll=2)
        def _grp(gi, pos):
            off = pl.multiple_of(gi * L, L)
            d16 = sup_dst[sb, pl.ds(off, L)]
            l16 = d16 - base
            m = jnp.logical_and(l16 >= 0, l16 < ROWS_PT)
            cnt = plsc.all_reduce_population_count(m)

            @pl.when(cnt[0] > 0)
            def _():
                s16 = sup_src[sb, pl.ds(off, L)]
                w16 = sup_w[sb, pl.ds(off, L)]
                excl = plsc.cumsum(m.astype(jnp.int32)) - m.astype(jnp.int32)
                slot = pos + excl
                plsc.store_scatter(pend_src, [slot], s16, mask=m)
                plsc.store_scatter(pend_w, [slot], w16, mask=m)
                plsc.store_scatter(pend_loc, [slot], l16, mask=m)

            pos2 = pos + cnt[0]

            @pl.when(pos2 >= CHUNK)
            def _():
                process_chunk()

            return jnp.where(pos2 >= CHUNK, pos2 - CHUNK, pos2)

        @pl.when(q + 1 < NSUP)
        def _():
            wait_staging(q + 1)

        return _grp

    pos_end = _block

    for g in range(CHUNK // L):
        idx = jnp.full((L,), g * L, jnp.int32) + iota
        m = idx >= pos_end
        plsc.store_scatter(pend_w, [idx], jnp.zeros((L,), jnp.float32),
                           mask=m)
        plsc.store_scatter(pend_loc, [idx], jnp.zeros((L,), jnp.int32),
                           mask=m)
        plsc.store_scatter(pend_src, [idx], jnp.zeros((L,), jnp.int32),
                           mask=m)
    process_chunk()

    last = NC * NS - 1

    @pl.when(t < last)
    def _():
        pltpu.sync_copy(acc, out_hbm.at[pl.ds(base, ROWS_PT)])

    @pl.when(t == last)
    def _():
        n = N_NODES - last * ROWS_PT
        pltpu.sync_copy(acc.at[pl.ds(0, n)], out_hbm.at[pl.ds(base, n)])


_MESH = plsc.VectorSubcoreMesh(core_axis_name="c", subcore_axis_name="s")


@functools.partial(
    pl.kernel,
    out_type=jax.ShapeDtypeStruct((N_NODES, D), jnp.float32),
    mesh=_MESH,
    compiler_params=pltpu.CompilerParams(needs_layout_passes=False),
    scratch_types=[
        pltpu.VMEM((ROWS_PT, D), jnp.float32),
        pltpu.VMEM((2, SUPER), jnp.int32),
        pltpu.VMEM((2, SUPER), jnp.int32),
        pltpu.VMEM((2, SUPER), jnp.float32),
        pltpu.VMEM((PEND,), jnp.int32),
        pltpu.VMEM((PEND,), jnp.float32),
        pltpu.VMEM((PEND,), jnp.int32),
        pltpu.VMEM((CHUNK, D), jnp.float32),
        pltpu.SemaphoreType.DMA,
        pltpu.SemaphoreType.DMA,
        pltpu.SemaphoreType.DMA,
    ],
)
def _hyperbolic_agg_sc(x_hbm, src_hbm, dst_hbm, w_hbm, out_hbm, *scratch):
    _sc_body(x_hbm, src_hbm, dst_hbm, w_hbm, out_hbm, *scratch)


def kernel(x, edge_index, edge_weight):
    src = edge_index[0].astype(jnp.int32)
    dst = edge_index[1].astype(jnp.int32)
    w = edge_weight.astype(jnp.float32)
    pad = E_PAD - src.shape[0]
    src = jnp.concatenate([src, jnp.zeros((pad,), jnp.int32)])
    dst = jnp.concatenate([dst, jnp.full((pad,), NC * NS * ROWS_PT,
                                         jnp.int32)])
    w = jnp.concatenate([w, jnp.zeros((pad,), jnp.float32)])
    return _hyperbolic_agg_sc(x, src, dst, w)


# speedup vs baseline: 1.5284x; 1.1537x over previous
import functools

import jax
import jax.numpy as jnp
from jax import lax
from jax.experimental import pallas as pl
from jax.experimental.pallas import tpu as pltpu
from jax.experimental.pallas import tpu_sc as plsc

N_NODES = 10000
N_EDGES = 160000
D = 256

NC = 2
NS = 16
L = 16
G = D // L

CHUNK = 128
PEND = 192
SUPER = 1024
NSUP = 160
E_PAD = NSUP * SUPER
GPB = SUPER // L

ROWS_PT = 320


def _sc_body(x_hbm, src_hbm, dst_hbm, w_hbm, out_hbm,
             acc, sup_src, sup_dst, sup_w, pend_src, pend_w, pend_loc,
             rows1, gsem, tsem0, tsem1):
    c = lax.axis_index("c")
    s = lax.axis_index("s")
    tsem = (tsem0, tsem1)
    t = c * NS + s
    base = t * ROWS_PT
    zeros = jnp.zeros((L,), jnp.float32)
    iota = lax.iota(jnp.int32, L)

    def staging_cps(q, sb):
        e0 = q * SUPER
        return (
            pltpu.make_async_copy(src_hbm.at[pl.ds(e0, SUPER)],
                                  sup_src.at[sb], tsem[sb]),
            pltpu.make_async_copy(dst_hbm.at[pl.ds(e0, SUPER)],
                                  sup_dst.at[sb], tsem[sb]),
            pltpu.make_async_copy(w_hbm.at[pl.ds(e0, SUPER)],
                                  sup_w.at[sb], tsem[sb]),
        )

    def fire_staging(q1):
        @pl.when((q1 & 1) == 0)
        def _():
            for cp in staging_cps(q1, 0):
                cp.start()

        @pl.when((q1 & 1) == 1)
        def _():
            for cp in staging_cps(q1, 1):
                cp.start()

    def wait_staging(q1):
        @pl.when((q1 & 1) == 0)
        def _():
            for cp in staging_cps(q1, 0):
                cp.wait()

        @pl.when((q1 & 1) == 1)
        def _():
            for cp in staging_cps(q1, 1):
                cp.wait()

    def process_chunk():
        pltpu.async_copy(x_hbm.at[pend_src.at[pl.ds(0, CHUNK)]],
                         rows1, gsem).wait()

        @pl.loop(0, CHUNK // L)
        def _grp(g):
            go = pl.multiple_of(g * L, L)
            w16 = pend_w[pl.ds(go, L)]
            l16 = pend_loc[pl.ds(go, L)]
            for k in range(L):
                wv = jnp.full((L,), w16[k], dtype=jnp.float32)
                lk = l16[k]
                e = go + k
                for j in range(G):
                    acc[lk, pl.ds(j * L, L)] = (
                        acc[lk, pl.ds(j * L, L)]
                        + rows1[e, pl.ds(j * L, L)] * wv)

        for g in range((PEND - CHUNK) // L):
            o_src = CHUNK + g * L
            o_dst = g * L
            pend_src[pl.ds(o_dst, L)] = pend_src[pl.ds(o_src, L)]
            pend_w[pl.ds(o_dst, L)] = pend_w[pl.ds(o_src, L)]
            pend_loc[pl.ds(o_dst, L)] = pend_loc[pl.ds(o_src, L)]

    @pl.loop(0, ROWS_PT)
    def _zero(r):
        for j in range(G):
            acc[r, pl.ds(j * L, L)] = zeros

    for cp in staging_cps(0, 0):
        cp.start()
    for cp in staging_cps(0, 0):
        cp.wait()

    @pl.loop(0, NSUP, init_carry=jnp.int32(0))
    def _block(q, pos_in):
        @pl.when(q + 1 < NSUP)
        def _():
            fire_staging(q + 1)

        sb = q & 1

        @pl.loop(0, GPB // 4, init_carry=pos_in)
        def _grp(gq, pos):
            for u in range(4):
                off = pl.multiple_of(gq * 4 * L, L) + u * L
                d16 = sup_dst[sb, pl.ds(off, L)]
                l16 = d16 - base
                m = jnp.logical_and(l16 >= 0, l16 < ROWS_PT)
                mi = m.astype(jnp.int32)
                cum = plsc.cumsum(mi)
                slot = pos + (cum - mi)
                s16 = sup_src[sb, pl.ds(off, L)]
                w16 = sup_w[sb, pl.ds(off, L)]
                plsc.store_scatter(pend_src, [slot], s16, mask=m)
                plsc.store_scatter(pend_w, [slot], w16, mask=m)
                plsc.store_scatter(pend_loc, [slot], l16, mask=m)
                pos = pos + cum[L - 1]

            @pl.when(pos >= CHUNK)
            def _():
                process_chunk()

            return jnp.where(pos >= CHUNK, pos - CHUNK, pos)

        @pl.when(q + 1 < NSUP)
        def _():
            wait_staging(q + 1)

        return _grp

    pos_end = _block

    for g in range(CHUNK // L):
        idx = jnp.full((L,), g * L, jnp.int32) + iota
        m = idx >= pos_end
        plsc.store_scatter(pend_w, [idx], jnp.zeros((L,), jnp.float32),
                           mask=m)
        plsc.store_scatter(pend_loc, [idx], jnp.zeros((L,), jnp.int32),
                           mask=m)
        plsc.store_scatter(pend_src, [idx], jnp.zeros((L,), jnp.int32),
                           mask=m)
    process_chunk()

    last = NC * NS - 1

    @pl.when(t < last)
    def _():
        pltpu.sync_copy(acc, out_hbm.at[pl.ds(base, ROWS_PT)])

    @pl.when(t == last)
    def _():
        n = N_NODES - last * ROWS_PT
        pltpu.sync_copy(acc.at[pl.ds(0, n)], out_hbm.at[pl.ds(base, n)])


_MESH = plsc.VectorSubcoreMesh(core_axis_name="c", subcore_axis_name="s")


@functools.partial(
    pl.kernel,
    out_type=jax.ShapeDtypeStruct((N_NODES, D), jnp.float32),
    mesh=_MESH,
    compiler_params=pltpu.CompilerParams(needs_layout_passes=False),
    scratch_types=[
        pltpu.VMEM((ROWS_PT, D), jnp.float32),
        pltpu.VMEM((2, SUPER), jnp.int32),
        pltpu.VMEM((2, SUPER), jnp.int32),
        pltpu.VMEM((2, SUPER), jnp.float32),
        pltpu.VMEM((PEND,), jnp.int32),
        pltpu.VMEM((PEND,), jnp.float32),
        pltpu.VMEM((PEND,), jnp.int32),
        pltpu.VMEM((CHUNK, D), jnp.float32),
        pltpu.SemaphoreType.DMA,
        pltpu.SemaphoreType.DMA,
        pltpu.SemaphoreType.DMA,
    ],
)
def _hyperbolic_agg_sc(x_hbm, src_hbm, dst_hbm, w_hbm, out_hbm, *scratch):
    _sc_body(x_hbm, src_hbm, dst_hbm, w_hbm, out_hbm, *scratch)


def kernel(x, edge_index, edge_weight):
    src = edge_index[0].astype(jnp.int32)
    dst = edge_index[1].astype(jnp.int32)
    w = edge_weight.astype(jnp.float32)
    pad = E_PAD - src.shape[0]
    src = jnp.concatenate([src, jnp.zeros((pad,), jnp.int32)])
    dst = jnp.concatenate([dst, jnp.full((pad,), NC * NS * ROWS_PT,
                                         jnp.int32)])
    w = jnp.concatenate([w, jnp.zeros((pad,), jnp.float32)])
    return _hyperbolic_agg_sc(x, src, dst, w)
